# Initial kernel scaffold; baseline (speedup 1.0000x reference)
import jax
import jax.numpy as jnp
from jax.experimental import pallas as pl


def kernel(pos, norm, edge_index, Wc1, bc1, W1, u1, c1, b1, W2, u2, c2, b2, W3, u3, c3, b3, W4, u4, c4, b4, Wc2, bc2):
    raise NotImplementedError("write your pallas kernel here")


# speedup vs baseline: 1.0383x; 1.0383x over previous
import functools

import jax
import jax.numpy as jnp
from jax.experimental import pallas as pl

N = 10000
E = 320000
H = 6
D = 128

TE = 1000


def _edge_math_body(xj_ref, xi_ref, u_ref, c_ref, W_ref, m_ref):
    xj = xj_ref[...]
    xi = xi_ref[...]
    q = (xj - xi) @ u_ref[...] + c_ref[...]
    q = q - jnp.max(q, axis=-1, keepdims=True)
    eq = jnp.exp(q)
    q = eq / jnp.sum(eq, axis=-1, keepdims=True)
    acc = jnp.zeros((TE, D), jnp.float32)
    for h in range(H):
        acc = acc + q[:, h:h + 1] * (xj @ W_ref[0, h])
    m_ref[...] = acc


def _edge_math(xj, xi, u, c, W):
    u_pad = jnp.zeros((D, 128), jnp.float32).at[:, :H].set(u)
    c_pad = jnp.full((128,), -1e30, jnp.float32).at[:H].set(c)
    Wr = W.reshape(D, H, D).transpose(1, 0, 2)
    grid = E // TE
    return pl.pallas_call(
        _edge_math_body,
        grid=(grid,),
        in_specs=[
            pl.BlockSpec((TE, D), lambda i: (i, 0)),
            pl.BlockSpec((TE, D), lambda i: (i, 0)),
            pl.BlockSpec((D, 128), lambda i: (0, 0)),
            pl.BlockSpec((128,), lambda i: (0,)),
            pl.BlockSpec((1, H, D, D), lambda i: (0, 0, 0, 0)),
        ],
        out_specs=pl.BlockSpec((TE, D), lambda i: (i, 0)),
        out_shape=jax.ShapeDtypeStruct((E, D), jnp.float32),
    )(xj, xi, u_pad, c_pad, Wr[None])


def _feast_layer(x, src, dst, W, u, c, b):
    xj = jnp.take(x, src, axis=0)
    xi = jnp.take(x, dst, axis=0)
    m = _edge_math(xj, xi, u, c, W)
    agg = jax.ops.segment_sum(m, dst, num_segments=N)
    cnt = jax.ops.segment_sum(jnp.ones((E, 1), jnp.float32), dst, num_segments=N)
    return agg / jnp.maximum(cnt, 1.0) + b


def kernel(pos, norm, edge_index, Wc1, bc1, W1, u1, c1, b1, W2, u2, c2, b2,
           W3, u3, c3, b3, W4, u4, c4, b4, Wc2, bc2):
    src = edge_index[0]
    dst = edge_index[1]
    x = jnp.concatenate([pos, norm], axis=1)
    x = jax.nn.relu(x @ Wc1.T + bc1)
    x = jax.nn.relu(_feast_layer(x, src, dst, W1, u1, c1, b1))
    x = jax.nn.relu(_feast_layer(x, src, dst, W2, u2, c2, b2))
    x = jax.nn.relu(_feast_layer(x, src, dst, W3, u3, c3, b3))
    x = jax.nn.relu(_feast_layer(x, src, dst, W4, u4, c4, b4))
    return x @ Wc2.T + bc2


# speedup vs baseline: 1.1698x; 1.1267x over previous
import functools

import jax
import jax.numpy as jnp
from jax import lax
from jax.experimental import pallas as pl
from jax.experimental.pallas import tpu as pltpu
from jax.experimental.pallas import tpu_sc as plsc

N = 10000
E = 320000
H = 6
D = 128
YD = H * D
D2 = D + 16
NC, NS = 2, 16
NW = NC * NS
EPW = E // NW
BLK = 16
NPAD = 10240
RB = 512
ZR = 64


def _prologue_body(xin_ref, Wc1_ref, b_ref, W_ref, u_ref, Y_ref, p_ref):
    x = jnp.maximum(xin_ref[...] @ Wc1_ref[...] + b_ref[...], 0.0)
    Y_ref[...] = x @ W_ref[...]
    p_ref[...] = x @ u_ref[...]


def _prologue(xin, Wc1p, b1p, W, u16):
    return pl.pallas_call(
        _prologue_body,
        grid=(NPAD // RB,),
        in_specs=[
            pl.BlockSpec((RB, D), lambda i: (i, 0)),
            pl.BlockSpec((D, D), lambda i: (0, 0)),
            pl.BlockSpec((1, D), lambda i: (0, 0)),
            pl.BlockSpec((D, YD), lambda i: (0, 0)),
            pl.BlockSpec((D, 16), lambda i: (0, 0)),
        ],
        out_specs=[
            pl.BlockSpec((RB, YD), lambda i: (i, 0)),
            pl.BlockSpec((RB, 16), lambda i: (i, 0)),
        ],
        out_shape=[
            jax.ShapeDtypeStruct((NPAD, YD), jnp.float32),
            jax.ShapeDtypeStruct((NPAD, 16), jnp.float32),
        ],
    )(xin, Wc1p, b1p, W, u16)


def _combine_body(a0_ref, a1_ref, b_ref, W_ref, u_ref, Y_ref, p_ref):
    a = a0_ref[...] + a1_ref[...]
    cnt = jnp.maximum(a[:, D:D + 1], 1.0)
    x = jnp.maximum(a[:, :D] / cnt + b_ref[...], 0.0)
    Y_ref[...] = x @ W_ref[...]
    p_ref[...] = x @ u_ref[...]


def _combine(agg0, agg1, bp, W, u16):
    return pl.pallas_call(
        _combine_body,
        grid=(NPAD // RB,),
        in_specs=[
            pl.BlockSpec((RB, D2), lambda i: (i, 0)),
            pl.BlockSpec((RB, D2), lambda i: (i, 0)),
            pl.BlockSpec((1, D), lambda i: (0, 0)),
            pl.BlockSpec((D, YD), lambda i: (0, 0)),
            pl.BlockSpec((D, 16), lambda i: (0, 0)),
        ],
        out_specs=[
            pl.BlockSpec((RB, YD), lambda i: (i, 0)),
            pl.BlockSpec((RB, 16), lambda i: (i, 0)),
        ],
        out_shape=[
            jax.ShapeDtypeStruct((NPAD, YD), jnp.float32),
            jax.ShapeDtypeStruct((NPAD, 16), jnp.float32),
        ],
    )(agg0, agg1, bp, W, u16)


def _epilogue_body(a0_ref, a1_ref, b_ref, Wc2_ref, bc2_ref, out_ref):
    a = a0_ref[...] + a1_ref[...]
    cnt = jnp.maximum(a[:, D:D + 1], 1.0)
    x = jnp.maximum(a[:, :D] / cnt + b_ref[...], 0.0)
    out_ref[...] = x @ Wc2_ref[...] + bc2_ref[...]


def _epilogue(agg0, agg1, bp, Wc2p, bc2p):
    return pl.pallas_call(
        _epilogue_body,
        grid=(NPAD // RB,),
        in_specs=[
            pl.BlockSpec((RB, D2), lambda i: (i, 0)),
            pl.BlockSpec((RB, D2), lambda i: (i, 0)),
            pl.BlockSpec((1, D), lambda i: (0, 0)),
            pl.BlockSpec((D, D), lambda i: (0, 0)),
            pl.BlockSpec((1, D), lambda i: (0, 0)),
        ],
        out_specs=pl.BlockSpec((RB, D), lambda i: (i, 0)),
        out_shape=jax.ShapeDtypeStruct((NPAD, D), jnp.float32),
    )(agg0, agg1, bp, Wc2p, bc2p)


def _sc_edge_layer(Y, p16, src, dst, c16, z64):
    mesh = plsc.VectorSubcoreMesh(core_axis_name="c", subcore_axis_name="s",
                                  num_cores=NC, num_subcores=NS)

    @functools.partial(
        pl.kernel,
        out_type=jax.ShapeDtypeStruct((NC, NPAD, D2), jnp.float32),
        mesh=mesh,
        compiler_params=pltpu.CompilerParams(needs_layout_passes=False,
                                             use_tc_tiling_on_sc=False),
        scratch_types=[
            pltpu.VMEM_SHARED((NPAD, D2), jnp.float32),
            pltpu.VMEM((BLK,), jnp.int32),
            pltpu.VMEM((BLK,), jnp.int32),
            pltpu.VMEM((BLK, 16), jnp.float32),
            pltpu.VMEM((BLK, 16), jnp.float32),
            pltpu.VMEM((BLK, YD), jnp.float32),
            pltpu.VMEM((BLK, D2), jnp.float32),
            pltpu.VMEM((16,), jnp.float32),
            pltpu.VMEM((ZR, D2), jnp.float32),
            pltpu.SemaphoreType.DMA,
        ],
    )
    def k(Y_h, p_h, src_h, dst_h, c_h, z_h, out_h,
          aggs, sidx, didx, ps, pd, yg, mb, cv, zb, sem):
        cid = lax.axis_index("c")
        sid = lax.axis_index("s")
        wid = sid * NC + cid

        pltpu.sync_copy(z_h, zb)
        rows_per_sub = NPAD // NS
        for t in range(rows_per_sub // ZR):
            pltpu.sync_copy(zb, aggs.at[pl.ds(sid * rows_per_sub + t * ZR, ZR)])
        pltpu.sync_copy(c_h, cv)
        plsc.subcore_barrier()

        cvec = cv[...]
        iota16 = lax.iota(jnp.int32, 16)
        ones16 = jnp.where(iota16 == 0, jnp.float32(1.0), jnp.float32(0.0))
        base0 = wid * EPW

        def blk(g, carry):
            base = base0 + g * BLK
            pltpu.sync_copy(src_h.at[pl.ds(base, BLK)], sidx)
            pltpu.sync_copy(dst_h.at[pl.ds(base, BLK)], didx)
            pltpu.async_copy(p_h.at[sidx], ps, sem).wait()
            pltpu.async_copy(p_h.at[didx], pd, sem).wait()
            pltpu.async_copy(Y_h.at[sidx], yg, sem).wait()
            qs = []
            for h in range(H):
                col = jnp.full((16,), h, jnp.int32)
                qs.append(plsc.load_gather(ps, [iota16, col])
                          - plsc.load_gather(pd, [iota16, col]) + cvec[h])
            mx = qs[0]
            for h in range(1, H):
                mx = jnp.maximum(mx, qs[h])
            es = [jnp.exp(q - mx) for q in qs]
            ssum = es[0]
            for h in range(1, H):
                ssum = ssum + es[h]
            ws = [e_ / ssum for e_ in es]
            for e in range(BLK):
                for j in range(D // 16):
                    acc = ws[0][e] * yg[e, pl.ds(j * 16, 16)]
                    for h in range(1, H):
                        acc = acc + ws[h][e] * yg[e, pl.ds(h * D + j * 16, 16)]
                    mb[e, pl.ds(j * 16, 16)] = acc
                mb[e, pl.ds(D, 16)] = ones16
            pltpu.sync_copy(mb, aggs.at[didx], add=True)
            return carry

        lax.fori_loop(0, EPW // BLK, blk, 0)
        plsc.subcore_barrier()
        pltpu.sync_copy(aggs.at[pl.ds(sid * rows_per_sub, rows_per_sub)],
                        out_h.at[cid, pl.ds(sid * rows_per_sub, rows_per_sub)])

    return k(Y, p16, src, dst, c16, z64)


def _pad_u(u):
    return jnp.zeros((D, 16), jnp.float32).at[:, :H].set(u)


def _pad_c(c):
    return jnp.full((16,), -1e30, jnp.float32).at[:H].set(c)


def kernel(pos, norm, edge_index, Wc1, bc1, W1, u1, c1, b1, W2, u2, c2, b2,
           W3, u3, c3, b3, W4, u4, c4, b4, Wc2, bc2):
    src = edge_index[0]
    dst = edge_index[1]
    z64 = jnp.zeros((ZR, D2), jnp.float32)

    xin = jnp.zeros((NPAD, D), jnp.float32)
    xin = xin.at[:N, :6].set(jnp.concatenate([pos, norm], axis=1))
    Wc1p = jnp.zeros((D, D), jnp.float32).at[:6].set(Wc1.T)
    Y, p16 = _prologue(xin, Wc1p, bc1[None], W1, _pad_u(u1))

    agg = _sc_edge_layer(Y, p16, src, dst, _pad_c(c1), z64)
    Y, p16 = _combine(agg[0], agg[1], b1[None], W2, _pad_u(u2))
    agg = _sc_edge_layer(Y, p16, src, dst, _pad_c(c2), z64)
    Y, p16 = _combine(agg[0], agg[1], b2[None], W3, _pad_u(u3))
    agg = _sc_edge_layer(Y, p16, src, dst, _pad_c(c3), z64)
    Y, p16 = _combine(agg[0], agg[1], b3[None], W4, _pad_u(u4))
    agg = _sc_edge_layer(Y, p16, src, dst, _pad_c(c4), z64)

    Wc2p = jnp.zeros((D, D), jnp.float32).at[:, :3].set(Wc2.T)
    bc2p = jnp.zeros((D,), jnp.float32).at[:3].set(bc2)
    out = _epilogue(agg[0], agg[1], b4[None], Wc2p, bc2p[None])
    return out[:N, :3]


# speedup vs baseline: 2.1152x; 1.8082x over previous
import functools

import jax
import jax.numpy as jnp
from jax import lax
from jax.experimental import pallas as pl
from jax.experimental.pallas import tpu as pltpu
from jax.experimental.pallas import tpu_sc as plsc

N = 10000
E = 320000
H = 6
D = 128
YD = H * D
D2 = D + 16
NC, NS = 2, 16
NW = NC * NS
BLK = 16
NPAD = 10240
EP = NW * 10240
RB = 512
ZR = 16


def _prologue_body(xin_ref, Wc1_ref, b_ref, W_ref, u_ref, Y_ref, p_ref):
    x = jnp.maximum(xin_ref[...] @ Wc1_ref[...] + b_ref[...], 0.0)
    Y_ref[...] = x @ W_ref[...]
    p_ref[...] = x @ u_ref[...]


def _prologue(xin, Wc1p, b1p, W, u16):
    return pl.pallas_call(
        _prologue_body,
        grid=(NPAD // RB,),
        in_specs=[
            pl.BlockSpec((RB, D), lambda i: (i, 0)),
            pl.BlockSpec((D, D), lambda i: (0, 0)),
            pl.BlockSpec((1, D), lambda i: (0, 0)),
            pl.BlockSpec((D, YD), lambda i: (0, 0)),
            pl.BlockSpec((D, 16), lambda i: (0, 0)),
        ],
        out_specs=[
            pl.BlockSpec((RB, YD), lambda i: (i, 0)),
            pl.BlockSpec((RB, 16), lambda i: (i, 0)),
        ],
        out_shape=[
            jax.ShapeDtypeStruct((NPAD, YD), jnp.float32),
            jax.ShapeDtypeStruct((NPAD, 16), jnp.float32),
        ],
    )(xin, Wc1p, b1p, W, u16)


def _combine_body(a0_ref, a1_ref, b_ref, W_ref, u_ref, Y_ref, p_ref):
    a = a0_ref[...] + a1_ref[...]
    cnt = jnp.maximum(a[:, D:D + 1], 1.0)
    x = jnp.maximum(a[:, :D] / cnt + b_ref[...], 0.0)
    Y_ref[...] = x @ W_ref[...]
    p_ref[...] = x @ u_ref[...]


def _combine(agg0, agg1, bp, W, u16):
    return pl.pallas_call(
        _combine_body,
        grid=(NPAD // RB,),
        in_specs=[
            pl.BlockSpec((RB, D2), lambda i: (i, 0)),
            pl.BlockSpec((RB, D2), lambda i: (i, 0)),
            pl.BlockSpec((1, D), lambda i: (0, 0)),
            pl.BlockSpec((D, YD), lambda i: (0, 0)),
            pl.BlockSpec((D, 16), lambda i: (0, 0)),
        ],
        out_specs=[
            pl.BlockSpec((RB, YD), lambda i: (i, 0)),
            pl.BlockSpec((RB, 16), lambda i: (i, 0)),
        ],
        out_shape=[
            jax.ShapeDtypeStruct((NPAD, YD), jnp.float32),
            jax.ShapeDtypeStruct((NPAD, 16), jnp.float32),
        ],
    )(agg0, agg1, bp, W, u16)


def _epilogue_body(a0_ref, a1_ref, b_ref, Wc2_ref, bc2_ref, out_ref):
    a = a0_ref[...] + a1_ref[...]
    cnt = jnp.maximum(a[:, D:D + 1], 1.0)
    x = jnp.maximum(a[:, :D] / cnt + b_ref[...], 0.0)
    out_ref[...] = x @ Wc2_ref[...] + bc2_ref[...]


def _epilogue(agg0, agg1, bp, Wc2p, bc2p):
    return pl.pallas_call(
        _epilogue_body,
        grid=(NPAD // RB,),
        in_specs=[
            pl.BlockSpec((RB, D2), lambda i: (i, 0)),
            pl.BlockSpec((RB, D2), lambda i: (i, 0)),
            pl.BlockSpec((1, D), lambda i: (0, 0)),
            pl.BlockSpec((D, D), lambda i: (0, 0)),
            pl.BlockSpec((1, D), lambda i: (0, 0)),
        ],
        out_specs=pl.BlockSpec((RB, D), lambda i: (i, 0)),
        out_shape=jax.ShapeDtypeStruct((NPAD, D), jnp.float32),
    )(agg0, agg1, bp, Wc2p, bc2p)


def _sc_edge_layer(Y, p16, src, dst, c16, z64):
    mesh = plsc.VectorSubcoreMesh(core_axis_name="c", subcore_axis_name="s",
                                  num_cores=NC, num_subcores=NS)
    nblk = EP // NW // BLK

    @functools.partial(
        pl.kernel,
        out_type=jax.ShapeDtypeStruct((NC, NPAD, D2), jnp.float32),
        mesh=mesh,
        compiler_params=pltpu.CompilerParams(needs_layout_passes=False,
                                             use_tc_tiling_on_sc=False),
        scratch_types=[
            pltpu.VMEM_SHARED((NPAD, D2), jnp.float32),
            [pltpu.VMEM((BLK,), jnp.int32)] * 2,
            [pltpu.VMEM((BLK,), jnp.int32)] * 2,
            [pltpu.VMEM((BLK,), jnp.int32)] * 2,
            [pltpu.VMEM((BLK, 16), jnp.float32)] * 2,
            [pltpu.VMEM((BLK, 16), jnp.float32)] * 2,
            [pltpu.VMEM((BLK, YD), jnp.float32)] * 2,
            [pltpu.VMEM((BLK, D2), jnp.float32)] * 2,
            pltpu.VMEM((16,), jnp.float32),
            pltpu.VMEM((H * 16,), jnp.float32),
            pltpu.VMEM((ZR, D2), jnp.float32),
            [pltpu.SemaphoreType.DMA] * 2,
            [pltpu.SemaphoreType.DMA] * 2,
        ],
    )
    def k(Y_h, p_h, src_h, dst_h, c_h, z_h, out_h,
          aggs, sidxs, didxs, didx_ss, pss, pds, ygs, mbs, cv, wbuf, zb,
          semg, sems):
        cid = lax.axis_index("c")
        sid = lax.axis_index("s")
        wid = sid * NC + cid

        pltpu.sync_copy(z_h, zb)
        rows_per_sub = NPAD // NS
        for t in range(rows_per_sub // ZR):
            pltpu.sync_copy(zb, aggs.at[pl.ds(sid * rows_per_sub + t * ZR, ZR)])
        pltpu.sync_copy(c_h, cv)
        plsc.subcore_barrier()

        cvec = cv[...]
        iota16 = lax.iota(jnp.int32, 16)
        ones16 = jnp.where(iota16 == 0, jnp.float32(1.0), jnp.float32(0.0))
        base0 = wid * (EP // NW)

        def issue(b, g):
            base = base0 + g * BLK
            pltpu.sync_copy(src_h.at[pl.ds(base, BLK)], sidxs[b])
            pltpu.sync_copy(dst_h.at[pl.ds(base, BLK)], didxs[b])
            pltpu.async_copy(p_h.at[sidxs[b]], pss[b], semg[b])
            pltpu.async_copy(p_h.at[didxs[b]], pds[b], semg[b])
            pltpu.async_copy(Y_h.at[sidxs[b]], ygs[b], semg[b])

        def drain_gathers(b):
            pltpu.make_async_copy(p_h.at[sidxs[b]], pss[b], semg[b]).wait()
            pltpu.make_async_copy(p_h.at[didxs[b]], pds[b], semg[b]).wait()
            pltpu.make_async_copy(Y_h.at[sidxs[b]], ygs[b], semg[b]).wait()

        def drain_scatter(b):
            pltpu.make_async_copy(mbs[b], aggs.at[didx_ss[b]], sems[b]).wait()

        def consume(b, g):
            @pl.when(g >= 2)
            def _():
                drain_scatter(b)
            drain_gathers(b)

            qs = []
            for h in range(H):
                col = jnp.full((16,), h, jnp.int32)
                qs.append(plsc.load_gather(pss[b], [iota16, col])
                          - plsc.load_gather(pds[b], [iota16, col])
                          + cvec[h])
            mx = qs[0]
            for h in range(1, H):
                mx = jnp.maximum(mx, qs[h])
            es = [jnp.exp(q - mx) for q in qs]
            ssum = es[0]
            for h in range(1, H):
                ssum = ssum + es[h]
            rs = 1.0 / ssum
            for h in range(H):
                wbuf[pl.ds(h * 16, 16)] = es[h] * rs
            for e in range(BLK):
                wv = [plsc.load_gather(wbuf, [jnp.full((16,), h * 16 + e,
                                                       jnp.int32)])
                      for h in range(H)]
                for j in range(D // 16):
                    acc = wv[0] * ygs[b][e, pl.ds(j * 16, 16)]
                    for h in range(1, H):
                        acc = acc + wv[h] * ygs[b][e, pl.ds(h * D + j * 16, 16)]
                    mbs[b][e, pl.ds(j * 16, 16)] = acc
                mbs[b][e, pl.ds(D, 16)] = ones16
            didx_ss[b][...] = didxs[b][...]
            pltpu.async_copy(mbs[b], aggs.at[didx_ss[b]], sems[b], add=True)

        issue(0, 0)

        def outer(t, carry):
            for b in range(2):
                g = 2 * t + b

                @pl.when(g + 1 < nblk)
                def _():
                    issue((b + 1) % 2, g + 1)
                consume(b, g)
            return carry

        lax.fori_loop(0, nblk // 2, outer, 0)
        drain_scatter(0)
        drain_scatter(1)
        plsc.subcore_barrier()
        pltpu.sync_copy(aggs.at[pl.ds(sid * rows_per_sub, rows_per_sub)],
                        out_h.at[cid, pl.ds(sid * rows_per_sub, rows_per_sub)])

    return k(Y, p16, src, dst, c16, z64)


def _pad_u(u):
    return jnp.zeros((D, 16), jnp.float32).at[:, :H].set(u)


def _pad_c(c):
    return jnp.full((16,), -1e30, jnp.float32).at[:H].set(c)


def kernel(pos, norm, edge_index, Wc1, bc1, W1, u1, c1, b1, W2, u2, c2, b2,
           W3, u3, c3, b3, W4, u4, c4, b4, Wc2, bc2):
    epad = jnp.full((EP - E,), NPAD - 1, jnp.int32)
    src = jnp.concatenate([edge_index[0], epad])
    dst = jnp.concatenate([edge_index[1], epad])
    z64 = jnp.zeros((ZR, D2), jnp.float32)

    xin = jnp.zeros((NPAD, D), jnp.float32)
    xin = xin.at[:N, :6].set(jnp.concatenate([pos, norm], axis=1))
    Wc1p = jnp.zeros((D, D), jnp.float32).at[:6].set(Wc1.T)
    Y, p16 = _prologue(xin, Wc1p, bc1[None], W1, _pad_u(u1))

    agg = _sc_edge_layer(Y, p16, src, dst, _pad_c(c1), z64)
    Y, p16 = _combine(agg[0], agg[1], b1[None], W2, _pad_u(u2))
    agg = _sc_edge_layer(Y, p16, src, dst, _pad_c(c2), z64)
    Y, p16 = _combine(agg[0], agg[1], b2[None], W3, _pad_u(u3))
    agg = _sc_edge_layer(Y, p16, src, dst, _pad_c(c3), z64)
    Y, p16 = _combine(agg[0], agg[1], b3[None], W4, _pad_u(u4))
    agg = _sc_edge_layer(Y, p16, src, dst, _pad_c(c4), z64)

    Wc2p = jnp.zeros((D, D), jnp.float32).at[:, :3].set(Wc2.T)
    bc2p = jnp.zeros((D,), jnp.float32).at[:3].set(bc2)
    out = _epilogue(agg[0], agg[1], b4[None], Wc2p, bc2p[None])
    return out[:N, :3]


# speedup vs baseline: 2.2415x; 1.0597x over previous
import functools

import jax
import jax.numpy as jnp
from jax import lax
from jax.experimental import pallas as pl
from jax.experimental.pallas import tpu as pltpu
from jax.experimental.pallas import tpu_sc as plsc

N = 10000
E = 320000
H = 6
D = 128
YD = H * D
D2 = D + 16
NC, NS = 2, 16
NW = NC * NS
BLK = 16
NPAD = 10240
EP = NW * 10240
RB = 512
ZR = 16


def _prologue_body(xin_ref, Wc1_ref, b_ref, W_ref, u_ref, Y_ref, p_ref):
    x = jnp.maximum(xin_ref[...] @ Wc1_ref[...] + b_ref[...], 0.0)
    Y_ref[...] = x @ W_ref[...]
    p_ref[...] = x @ u_ref[...]


def _prologue(xin, Wc1p, b1p, W, u16):
    return pl.pallas_call(
        _prologue_body,
        grid=(NPAD // RB,),
        in_specs=[
            pl.BlockSpec((RB, D), lambda i: (i, 0)),
            pl.BlockSpec((D, D), lambda i: (0, 0)),
            pl.BlockSpec((1, D), lambda i: (0, 0)),
            pl.BlockSpec((D, YD), lambda i: (0, 0)),
            pl.BlockSpec((D, 16), lambda i: (0, 0)),
        ],
        out_specs=[
            pl.BlockSpec((RB, YD), lambda i: (i, 0)),
            pl.BlockSpec((RB, 16), lambda i: (i, 0)),
        ],
        out_shape=[
            jax.ShapeDtypeStruct((NPAD, YD), jnp.float32),
            jax.ShapeDtypeStruct((NPAD, 16), jnp.float32),
        ],
    )(xin, Wc1p, b1p, W, u16)


def _combine_body(a0_ref, a1_ref, b_ref, W_ref, u_ref, Y_ref, p_ref):
    a = a0_ref[...] + a1_ref[...]
    cnt = jnp.maximum(a[:, D:D + 1], 1.0)
    x = jnp.maximum(a[:, :D] / cnt + b_ref[...], 0.0)
    Y_ref[...] = x @ W_ref[...]
    p_ref[...] = x @ u_ref[...]


def _combine(agg0, agg1, bp, W, u16):
    return pl.pallas_call(
        _combine_body,
        grid=(NPAD // RB,),
        in_specs=[
            pl.BlockSpec((RB, D2), lambda i: (i, 0)),
            pl.BlockSpec((RB, D2), lambda i: (i, 0)),
            pl.BlockSpec((1, D), lambda i: (0, 0)),
            pl.BlockSpec((D, YD), lambda i: (0, 0)),
            pl.BlockSpec((D, 16), lambda i: (0, 0)),
        ],
        out_specs=[
            pl.BlockSpec((RB, YD), lambda i: (i, 0)),
            pl.BlockSpec((RB, 16), lambda i: (i, 0)),
        ],
        out_shape=[
            jax.ShapeDtypeStruct((NPAD, YD), jnp.float32),
            jax.ShapeDtypeStruct((NPAD, 16), jnp.float32),
        ],
    )(agg0, agg1, bp, W, u16)


def _epilogue_body(a0_ref, a1_ref, b_ref, Wc2_ref, bc2_ref, out_ref):
    a = a0_ref[...] + a1_ref[...]
    cnt = jnp.maximum(a[:, D:D + 1], 1.0)
    x = jnp.maximum(a[:, :D] / cnt + b_ref[...], 0.0)
    out_ref[...] = x @ Wc2_ref[...] + bc2_ref[...]


def _epilogue(agg0, agg1, bp, Wc2p, bc2p):
    return pl.pallas_call(
        _epilogue_body,
        grid=(NPAD // RB,),
        in_specs=[
            pl.BlockSpec((RB, D2), lambda i: (i, 0)),
            pl.BlockSpec((RB, D2), lambda i: (i, 0)),
            pl.BlockSpec((1, D), lambda i: (0, 0)),
            pl.BlockSpec((D, D), lambda i: (0, 0)),
            pl.BlockSpec((1, D), lambda i: (0, 0)),
        ],
        out_specs=pl.BlockSpec((RB, D), lambda i: (i, 0)),
        out_shape=jax.ShapeDtypeStruct((NPAD, D), jnp.float32),
    )(agg0, agg1, bp, Wc2p, bc2p)


CHB = 16
CHE = CHB * BLK


def _sc_edge_layer(Y, p16, src, dst, c16, z64):
    mesh = plsc.VectorSubcoreMesh(core_axis_name="c", subcore_axis_name="s",
                                  num_cores=NC, num_subcores=NS)
    nblk = EP // NW // BLK
    nch = nblk // CHB

    @functools.partial(
        pl.kernel,
        out_type=jax.ShapeDtypeStruct((NC, NPAD, D2), jnp.float32),
        mesh=mesh,
        compiler_params=pltpu.CompilerParams(needs_layout_passes=False,
                                             use_tc_tiling_on_sc=False),
        scratch_types=[
            pltpu.VMEM_SHARED((NPAD, D2), jnp.float32),
            [pltpu.VMEM((CHE,), jnp.int32)] * 2,
            [pltpu.VMEM((CHE,), jnp.int32)] * 2,
            [pltpu.VMEM((BLK,), jnp.int32)] * 2,
            [pltpu.VMEM((BLK, 16), jnp.float32)] * 2,
            [pltpu.VMEM((BLK, 16), jnp.float32)] * 2,
            [pltpu.VMEM((BLK, YD), jnp.float32)] * 2,
            [pltpu.VMEM((BLK, D2), jnp.float32)] * 2,
            pltpu.VMEM((16,), jnp.float32),
            pltpu.VMEM((H * 16,), jnp.float32),
            pltpu.VMEM((ZR, D2), jnp.float32),
            [pltpu.SemaphoreType.DMA] * 2,
            [pltpu.SemaphoreType.DMA] * 2,
            pltpu.SemaphoreType.DMA,
        ],
    )
    def k(Y_h, p_h, src_h, dst_h, c_h, z_h, out_h,
          aggs, scs, scd, didx_ss, pss, pds, ygs, mbs, cv, wbuf, zb,
          semg, sems, semi):
        cid = lax.axis_index("c")
        sid = lax.axis_index("s")
        wid = sid * NC + cid

        pltpu.sync_copy(z_h, zb)
        rows_per_sub = NPAD // NS
        for t in range(rows_per_sub // ZR):
            pltpu.sync_copy(zb, aggs.at[pl.ds(sid * rows_per_sub + t * ZR, ZR)])
        pltpu.sync_copy(c_h, cv)
        plsc.subcore_barrier()

        cvec = cv[...]
        iota16 = lax.iota(jnp.int32, 16)
        ones16 = jnp.where(iota16 == 0, jnp.float32(1.0), jnp.float32(0.0))
        for b in range(2):
            for e in range(BLK):
                mbs[b][e, pl.ds(D, 16)] = ones16
        base0 = wid * (EP // NW)

        def issue_idx(s, ci):
            base = base0 + ci * CHE
            pltpu.async_copy(src_h.at[pl.ds(base, CHE)], scs[s], semi)
            pltpu.async_copy(dst_h.at[pl.ds(base, CHE)], scd[s], semi)

        def drain_idx(s, ci):
            base = base0 + ci * CHE
            pltpu.make_async_copy(src_h.at[pl.ds(base, CHE)], scs[s],
                                  semi).wait()
            pltpu.make_async_copy(dst_h.at[pl.ds(base, CHE)], scd[s],
                                  semi).wait()

        def issue(b, s, local):
            sref = scs[s].at[pl.ds(local * BLK, BLK)]
            dref = scd[s].at[pl.ds(local * BLK, BLK)]
            pltpu.async_copy(p_h.at[sref], pss[b], semg[b])
            pltpu.async_copy(p_h.at[dref], pds[b], semg[b])
            pltpu.async_copy(Y_h.at[sref], ygs[b], semg[b])

        def drain_gathers(b, s, local):
            sref = scs[s].at[pl.ds(local * BLK, BLK)]
            dref = scd[s].at[pl.ds(local * BLK, BLK)]
            pltpu.make_async_copy(p_h.at[sref], pss[b], semg[b]).wait()
            pltpu.make_async_copy(p_h.at[dref], pds[b], semg[b]).wait()
            pltpu.make_async_copy(Y_h.at[sref], ygs[b], semg[b]).wait()

        def drain_scatter(b):
            pltpu.make_async_copy(mbs[b], aggs.at[didx_ss[b]], sems[b]).wait()

        def consume(b, s, local, g):
            @pl.when(g >= 2)
            def _():
                drain_scatter(b)
            drain_gathers(b, s, local)

            qs = []
            for h in range(H):
                col = jnp.full((16,), h, jnp.int32)
                qs.append(plsc.load_gather(pss[b], [iota16, col])
                          - plsc.load_gather(pds[b], [iota16, col])
                          + cvec[h])
            mx = qs[0]
            for h in range(1, H):
                mx = jnp.maximum(mx, qs[h])
            es = [jnp.exp(q - mx) for q in qs]
            ssum = es[0]
            for h in range(1, H):
                ssum = ssum + es[h]
            rs = 1.0 / ssum
            for h in range(H):
                wbuf[pl.ds(h * 16, 16)] = es[h] * rs
            for e in range(BLK):
                wv = [plsc.load_gather(wbuf, [jnp.full((16,), h * 16 + e,
                                                       jnp.int32)])
                      for h in range(H)]
                for j in range(D // 16):
                    acc = wv[0] * ygs[b][e, pl.ds(j * 16, 16)]
                    for h in range(1, H):
                        acc = acc + wv[h] * ygs[b][e, pl.ds(h * D + j * 16, 16)]
                    mbs[b][e, pl.ds(j * 16, 16)] = acc
            didx_ss[b][...] = scd[s][pl.ds(local * BLK, BLK)]
            pltpu.async_copy(mbs[b], aggs.at[didx_ss[b]], sems[b], add=True)

        def chunk_body(s, ci):
            drain_idx(s, ci)

            @pl.when(ci + 1 < nch)
            def _():
                issue_idx((s + 1) % 2, ci + 1)
            issue(0, s, 0)

            def pair(t, carry):
                for b in range(2):
                    local = 2 * t + b

                    @pl.when(local + 1 < CHB)
                    def _():
                        issue((b + 1) % 2, s, local + 1)
                    consume(b, s, local, ci * CHB + local)
                return carry

            lax.fori_loop(0, CHB // 2, pair, 0)

        issue_idx(0, 0)

        def outer(cp, carry):
            for s in range(2):
                chunk_body(s, 2 * cp + s)
            return carry

        lax.fori_loop(0, nch // 2, outer, 0)
        drain_scatter(0)
        drain_scatter(1)
        plsc.subcore_barrier()
        pltpu.sync_copy(aggs.at[pl.ds(sid * rows_per_sub, rows_per_sub)],
                        out_h.at[cid, pl.ds(sid * rows_per_sub, rows_per_sub)])

    return k(Y, p16, src, dst, c16, z64)


def _pad_u(u):
    return jnp.zeros((D, 16), jnp.float32).at[:, :H].set(u)


def _pad_c(c):
    return jnp.full((16,), -1e30, jnp.float32).at[:H].set(c)


def kernel(pos, norm, edge_index, Wc1, bc1, W1, u1, c1, b1, W2, u2, c2, b2,
           W3, u3, c3, b3, W4, u4, c4, b4, Wc2, bc2):
    epad = jnp.full((EP - E,), NPAD - 1, jnp.int32)
    src = jnp.concatenate([edge_index[0], epad])
    dst = jnp.concatenate([edge_index[1], epad])
    z64 = jnp.zeros((ZR, D2), jnp.float32)

    xin = jnp.zeros((NPAD, D), jnp.float32)
    xin = xin.at[:N, :6].set(jnp.concatenate([pos, norm], axis=1))
    Wc1p = jnp.zeros((D, D), jnp.float32).at[:6].set(Wc1.T)
    Y, p16 = _prologue(xin, Wc1p, bc1[None], W1, _pad_u(u1))

    agg = _sc_edge_layer(Y, p16, src, dst, _pad_c(c1), z64)
    Y, p16 = _combine(agg[0], agg[1], b1[None], W2, _pad_u(u2))
    agg = _sc_edge_layer(Y, p16, src, dst, _pad_c(c2), z64)
    Y, p16 = _combine(agg[0], agg[1], b2[None], W3, _pad_u(u3))
    agg = _sc_edge_layer(Y, p16, src, dst, _pad_c(c3), z64)
    Y, p16 = _combine(agg[0], agg[1], b3[None], W4, _pad_u(u4))
    agg = _sc_edge_layer(Y, p16, src, dst, _pad_c(c4), z64)

    Wc2p = jnp.zeros((D, D), jnp.float32).at[:, :3].set(Wc2.T)
    bc2p = jnp.zeros((D,), jnp.float32).at[:3].set(bc2)
    out = _epilogue(agg[0], agg[1], b4[None], Wc2p, bc2p[None])
    return out[:N, :3]


# speedup vs baseline: 2.2822x; 1.0182x over previous
import functools

import jax
import jax.numpy as jnp
from jax import lax
from jax.experimental import pallas as pl
from jax.experimental.pallas import tpu as pltpu
from jax.experimental.pallas import tpu_sc as plsc

N = 10000
E = 320000
H = 6
D = 128
YD = H * D
D2 = D + 16
NC, NS = 2, 16
NW = NC * NS
BLK = 16
NPAD = 10240
EP = NW * 10240
RB = 512
ZR = 8


def _prologue_body(xin_ref, Wc1_ref, b_ref, W_ref, u_ref, YP_ref, p_ref):
    x = jnp.maximum(xin_ref[...] @ Wc1_ref[...] + b_ref[...], 0.0)
    p = x @ u_ref[...]
    YP_ref[...] = jnp.concatenate([x @ W_ref[...], p], axis=-1)
    p_ref[...] = p


def _prologue(xin, Wc1p, b1p, W, u16):
    return pl.pallas_call(
        _prologue_body,
        grid=(NPAD // RB,),
        in_specs=[
            pl.BlockSpec((RB, D), lambda i: (i, 0)),
            pl.BlockSpec((D, D), lambda i: (0, 0)),
            pl.BlockSpec((1, D), lambda i: (0, 0)),
            pl.BlockSpec((D, YD), lambda i: (0, 0)),
            pl.BlockSpec((D, 16), lambda i: (0, 0)),
        ],
        out_specs=[
            pl.BlockSpec((RB, YD + 16), lambda i: (i, 0)),
            pl.BlockSpec((RB, 16), lambda i: (i, 0)),
        ],
        out_shape=[
            jax.ShapeDtypeStruct((NPAD, YD + 16), jnp.float32),
            jax.ShapeDtypeStruct((NPAD, 16), jnp.float32),
        ],
    )(xin, Wc1p, b1p, W, u16)


def _combine_body(a0_ref, a1_ref, b_ref, W_ref, u_ref, YP_ref, p_ref):
    a = a0_ref[...] + a1_ref[...]
    cnt = jnp.maximum(a[:, D:D + 1], 1.0)
    x = jnp.maximum(a[:, :D] / cnt + b_ref[...], 0.0)
    p = x @ u_ref[...]
    YP_ref[...] = jnp.concatenate([x @ W_ref[...], p], axis=-1)
    p_ref[...] = p


def _combine(agg0, agg1, bp, W, u16):
    return pl.pallas_call(
        _combine_body,
        grid=(NPAD // RB,),
        in_specs=[
            pl.BlockSpec((RB, D2), lambda i: (i, 0)),
            pl.BlockSpec((RB, D2), lambda i: (i, 0)),
            pl.BlockSpec((1, D), lambda i: (0, 0)),
            pl.BlockSpec((D, YD), lambda i: (0, 0)),
            pl.BlockSpec((D, 16), lambda i: (0, 0)),
        ],
        out_specs=[
            pl.BlockSpec((RB, YD + 16), lambda i: (i, 0)),
            pl.BlockSpec((RB, 16), lambda i: (i, 0)),
        ],
        out_shape=[
            jax.ShapeDtypeStruct((NPAD, YD + 16), jnp.float32),
            jax.ShapeDtypeStruct((NPAD, 16), jnp.float32),
        ],
    )(agg0, agg1, bp, W, u16)


def _epilogue_body(a0_ref, a1_ref, b_ref, Wc2_ref, bc2_ref, out_ref):
    a = a0_ref[...] + a1_ref[...]
    cnt = jnp.maximum(a[:, D:D + 1], 1.0)
    x = jnp.maximum(a[:, :D] / cnt + b_ref[...], 0.0)
    out_ref[...] = x @ Wc2_ref[...] + bc2_ref[...]


def _epilogue(agg0, agg1, bp, Wc2p, bc2p):
    return pl.pallas_call(
        _epilogue_body,
        grid=(NPAD // RB,),
        in_specs=[
            pl.BlockSpec((RB, D2), lambda i: (i, 0)),
            pl.BlockSpec((RB, D2), lambda i: (i, 0)),
            pl.BlockSpec((1, D), lambda i: (0, 0)),
            pl.BlockSpec((D, D), lambda i: (0, 0)),
            pl.BlockSpec((1, D), lambda i: (0, 0)),
        ],
        out_specs=pl.BlockSpec((RB, D), lambda i: (i, 0)),
        out_shape=jax.ShapeDtypeStruct((NPAD, D), jnp.float32),
    )(agg0, agg1, bp, Wc2p, bc2p)


CHB = 16
CHE = CHB * BLK
YPD = YD + 16


def _sc_edge_layer(YP, p16, src, dst, c16, z64):
    mesh = plsc.VectorSubcoreMesh(core_axis_name="c", subcore_axis_name="s",
                                  num_cores=NC, num_subcores=NS)
    nblk = EP // NW // BLK
    nch = nblk // CHB

    @functools.partial(
        pl.kernel,
        out_type=jax.ShapeDtypeStruct((NC, NPAD, D2), jnp.float32),
        mesh=mesh,
        compiler_params=pltpu.CompilerParams(needs_layout_passes=False,
                                             use_tc_tiling_on_sc=False),
        scratch_types=[
            pltpu.VMEM_SHARED((NPAD, D2), jnp.float32),
            [pltpu.VMEM((CHE,), jnp.int32)] * 2,
            [pltpu.VMEM((CHE,), jnp.int32)] * 2,
            pltpu.VMEM((BLK,), jnp.int32),
            [pltpu.VMEM((CHE, 16), jnp.float32)] * 2,
            [pltpu.VMEM((BLK, YPD), jnp.float32)] * 2,
            pltpu.VMEM((BLK, D2), jnp.float32),
            pltpu.VMEM((16,), jnp.float32),
            pltpu.VMEM((H * 16,), jnp.float32),
            pltpu.VMEM((ZR, D2), jnp.float32),
            [pltpu.SemaphoreType.DMA] * 2,
            pltpu.SemaphoreType.DMA,
            pltpu.SemaphoreType.DMA,
            [pltpu.SemaphoreType.DMA] * 2,
        ],
    )
    def k(yp_h, p_h, src_h, dst_h, c_h, z_h, out_h,
          aggs, scs, scd, didx_s, pdc, yps, mb, cv, wbuf, zb,
          semg, sems, semi, semp):
        cid = lax.axis_index("c")
        sid = lax.axis_index("s")
        wid = sid * NC + cid

        pltpu.sync_copy(z_h, zb)
        rows_per_sub = NPAD // NS
        for t in range(rows_per_sub // ZR):
            pltpu.sync_copy(zb, aggs.at[pl.ds(sid * rows_per_sub + t * ZR, ZR)])
        pltpu.sync_copy(c_h, cv)
        plsc.subcore_barrier()

        cvec = cv[...]
        iota16 = lax.iota(jnp.int32, 16)
        ones16 = jnp.where(iota16 == 0, jnp.float32(1.0), jnp.float32(0.0))
        for e in range(BLK):
            mb[e, pl.ds(D, 16)] = ones16
        base0 = wid * (EP // NW)

        def issue_idx(s, ci):
            base = base0 + ci * CHE
            pltpu.async_copy(src_h.at[pl.ds(base, CHE)], scs[s], semi)
            pltpu.async_copy(dst_h.at[pl.ds(base, CHE)], scd[s], semi)

        def drain_idx(s, ci):
            base = base0 + ci * CHE
            pltpu.make_async_copy(src_h.at[pl.ds(base, CHE)], scs[s],
                                  semi).wait()
            pltpu.make_async_copy(dst_h.at[pl.ds(base, CHE)], scd[s],
                                  semi).wait()

        def issue_pd(s):
            for hf in range(2):
                pltpu.async_copy(p_h.at[scd[s].at[pl.ds(hf * 128, 128)]],
                                 pdc[s].at[pl.ds(hf * 128, 128)], semp[s])

        def drain_pd(s):
            for hf in range(2):
                pltpu.make_async_copy(p_h.at[scd[s].at[pl.ds(hf * 128, 128)]],
                                      pdc[s].at[pl.ds(hf * 128, 128)],
                                      semp[s]).wait()

        def issue(b, s, local):
            sref = scs[s].at[pl.ds(local * BLK, BLK)]
            pltpu.async_copy(yp_h.at[sref], yps[b], semg[b])

        def drain_gathers(b, s, local):
            sref = scs[s].at[pl.ds(local * BLK, BLK)]
            pltpu.make_async_copy(yp_h.at[sref], yps[b], semg[b]).wait()

        def drain_scatter():
            pltpu.make_async_copy(mb, aggs.at[didx_s], sems).wait()

        def consume(b, s, local, g):
            drain_gathers(b, s, local)

            qs = []
            for h in range(H):
                colp = jnp.full((16,), YD + h, jnp.int32)
                colq = jnp.full((16,), h, jnp.int32)
                rows = local * BLK + iota16
                qs.append(plsc.load_gather(yps[b], [iota16, colp])
                          - plsc.load_gather(pdc[s], [rows, colq])
                          + cvec[h])
            mx = qs[0]
            for h in range(1, H):
                mx = jnp.maximum(mx, qs[h])
            es = [jnp.exp(q - mx) for q in qs]
            ssum = es[0]
            for h in range(1, H):
                ssum = ssum + es[h]
            rs = 1.0 / ssum
            for h in range(H):
                wbuf[pl.ds(h * 16, 16)] = es[h] * rs

            @pl.when(g >= 1)
            def _():
                drain_scatter()
            for e in range(BLK):
                wv = [plsc.load_gather(wbuf, [jnp.full((16,), h * 16 + e,
                                                       jnp.int32)])
                      for h in range(H)]
                for j in range(D // 16):
                    acc = wv[0] * yps[b][e, pl.ds(j * 16, 16)]
                    for h in range(1, H):
                        acc = acc + wv[h] * yps[b][e, pl.ds(h * D + j * 16, 16)]
                    mb[e, pl.ds(j * 16, 16)] = acc
            didx_s[...] = scd[s][pl.ds(local * BLK, BLK)]
            pltpu.async_copy(mb, aggs.at[didx_s], sems, add=True)

        def chunk_body(s, ci):
            drain_idx(s, ci)

            @pl.when(ci + 1 < nch)
            def _():
                issue_idx((s + 1) % 2, ci + 1)
            issue_pd(s)
            issue(0, s, 0)
            drain_pd(s)

            def pair(t, carry):
                for b in range(2):
                    local = 2 * t + b

                    @pl.when(local + 1 < CHB)
                    def _():
                        issue((b + 1) % 2, s, local + 1)
                    consume(b, s, local, ci * CHB + local)
                return carry

            lax.fori_loop(0, CHB // 2, pair, 0)

        issue_idx(0, 0)

        def outer(cp, carry):
            for s in range(2):
                chunk_body(s, 2 * cp + s)
            return carry

        lax.fori_loop(0, nch // 2, outer, 0)
        drain_scatter()
        plsc.subcore_barrier()
        pltpu.sync_copy(aggs.at[pl.ds(sid * rows_per_sub, rows_per_sub)],
                        out_h.at[cid, pl.ds(sid * rows_per_sub, rows_per_sub)])

    return k(YP, p16, src, dst, c16, z64)


def _pad_u(u):
    return jnp.zeros((D, 16), jnp.float32).at[:, :H].set(u)


def _pad_c(c):
    return jnp.full((16,), -1e30, jnp.float32).at[:H].set(c)


def kernel(pos, norm, edge_index, Wc1, bc1, W1, u1, c1, b1, W2, u2, c2, b2,
           W3, u3, c3, b3, W4, u4, c4, b4, Wc2, bc2):
    epad = jnp.full((EP - E,), NPAD - 1, jnp.int32)
    src = jnp.concatenate([edge_index[0], epad])
    dst = jnp.concatenate([edge_index[1], epad])
    z64 = jnp.zeros((ZR, D2), jnp.float32)

    xin = jnp.zeros((NPAD, D), jnp.float32)
    xin = xin.at[:N, :6].set(jnp.concatenate([pos, norm], axis=1))
    Wc1p = jnp.zeros((D, D), jnp.float32).at[:6].set(Wc1.T)
    Y, p16 = _prologue(xin, Wc1p, bc1[None], W1, _pad_u(u1))

    agg = _sc_edge_layer(Y, p16, src, dst, _pad_c(c1), z64)
    Y, p16 = _combine(agg[0], agg[1], b1[None], W2, _pad_u(u2))
    agg = _sc_edge_layer(Y, p16, src, dst, _pad_c(c2), z64)
    Y, p16 = _combine(agg[0], agg[1], b2[None], W3, _pad_u(u3))
    agg = _sc_edge_layer(Y, p16, src, dst, _pad_c(c3), z64)
    Y, p16 = _combine(agg[0], agg[1], b3[None], W4, _pad_u(u4))
    agg = _sc_edge_layer(Y, p16, src, dst, _pad_c(c4), z64)

    Wc2p = jnp.zeros((D, D), jnp.float32).at[:, :3].set(Wc2.T)
    bc2p = jnp.zeros((D,), jnp.float32).at[:3].set(bc2)
    out = _epilogue(agg[0], agg[1], b4[None], Wc2p, bc2p[None])
    return out[:N, :3]


# speedup vs baseline: 3.5873x; 1.5718x over previous
import functools

import jax
import jax.numpy as jnp
from jax import lax
from jax.experimental import pallas as pl
from jax.experimental.pallas import tpu as pltpu
from jax.experimental.pallas import tpu_sc as plsc

N = 10000
E = 320000
H = 6
D = 128
DH = D // 2
YPC = H * DH + 16
D2 = DH + 16
NC, NS = 2, 16
NW = NC * NS
BLK = 16
NBUF = 4
NPAD = 10240
EP = NW * 10240
RB = 512
ZR = 8
CHB = 32
CHE = CHB * BLK


def _dense_body(x, Wa_ref, Wb_ref, u_ref, YP_ref, p_ref):
    p = x @ u_ref[...]
    YP_ref[0] = jnp.concatenate([x @ Wa_ref[...], p], axis=-1)
    YP_ref[1] = jnp.concatenate([x @ Wb_ref[...], p], axis=-1)
    p_ref[...] = p


def _prologue_body(xin_ref, Wc1_ref, b_ref, Wa_ref, Wb_ref, u_ref,
                   YP_ref, p_ref):
    x = jnp.maximum(xin_ref[...] @ Wc1_ref[...] + b_ref[...], 0.0)
    _dense_body(x, Wa_ref, Wb_ref, u_ref, YP_ref, p_ref)


def _prologue(xin, Wc1p, b1p, Wa, Wb, u16):
    return pl.pallas_call(
        _prologue_body,
        grid=(NPAD // RB,),
        in_specs=[
            pl.BlockSpec((RB, D), lambda i: (i, 0)),
            pl.BlockSpec((D, D), lambda i: (0, 0)),
            pl.BlockSpec((1, D), lambda i: (0, 0)),
            pl.BlockSpec((D, H * DH), lambda i: (0, 0)),
            pl.BlockSpec((D, H * DH), lambda i: (0, 0)),
            pl.BlockSpec((D, 16), lambda i: (0, 0)),
        ],
        out_specs=[
            pl.BlockSpec((NC, RB, YPC), lambda i: (0, i, 0)),
            pl.BlockSpec((RB, 16), lambda i: (i, 0)),
        ],
        out_shape=[
            jax.ShapeDtypeStruct((NC, NPAD, YPC), jnp.float32),
            jax.ShapeDtypeStruct((NPAD, 16), jnp.float32),
        ],
    )(xin, Wc1p, b1p, Wa, Wb, u16)


def _combine_body(a0_ref, a1_ref, b_ref, Wa_ref, Wb_ref, u_ref,
                  YP_ref, p_ref):
    cnt = jnp.maximum(a0_ref[:, DH:DH + 1], 1.0)
    feat = jnp.concatenate([a0_ref[:, :DH], a1_ref[:, :DH]], axis=-1)
    x = jnp.maximum(feat / cnt + b_ref[...], 0.0)
    _dense_body(x, Wa_ref, Wb_ref, u_ref, YP_ref, p_ref)


def _combine(agg0, agg1, bp, Wa, Wb, u16):
    return pl.pallas_call(
        _combine_body,
        grid=(NPAD // RB,),
        in_specs=[
            pl.BlockSpec((RB, D2), lambda i: (i, 0)),
            pl.BlockSpec((RB, D2), lambda i: (i, 0)),
            pl.BlockSpec((1, D), lambda i: (0, 0)),
            pl.BlockSpec((D, H * DH), lambda i: (0, 0)),
            pl.BlockSpec((D, H * DH), lambda i: (0, 0)),
            pl.BlockSpec((D, 16), lambda i: (0, 0)),
        ],
        out_specs=[
            pl.BlockSpec((NC, RB, YPC), lambda i: (0, i, 0)),
            pl.BlockSpec((RB, 16), lambda i: (i, 0)),
        ],
        out_shape=[
            jax.ShapeDtypeStruct((NC, NPAD, YPC), jnp.float32),
            jax.ShapeDtypeStruct((NPAD, 16), jnp.float32),
        ],
    )(agg0, agg1, bp, Wa, Wb, u16)


def _epilogue_body(a0_ref, a1_ref, b_ref, Wc2_ref, bc2_ref, out_ref):
    cnt = jnp.maximum(a0_ref[:, DH:DH + 1], 1.0)
    feat = jnp.concatenate([a0_ref[:, :DH], a1_ref[:, :DH]], axis=-1)
    x = jnp.maximum(feat / cnt + b_ref[...], 0.0)
    out_ref[...] = x @ Wc2_ref[...] + bc2_ref[...]


def _epilogue(agg0, agg1, bp, Wc2p, bc2p):
    return pl.pallas_call(
        _epilogue_body,
        grid=(NPAD // RB,),
        in_specs=[
            pl.BlockSpec((RB, D2), lambda i: (i, 0)),
            pl.BlockSpec((RB, D2), lambda i: (i, 0)),
            pl.BlockSpec((1, D), lambda i: (0, 0)),
            pl.BlockSpec((D, D), lambda i: (0, 0)),
            pl.BlockSpec((1, D), lambda i: (0, 0)),
        ],
        out_specs=pl.BlockSpec((RB, D), lambda i: (i, 0)),
        out_shape=jax.ShapeDtypeStruct((NPAD, D), jnp.float32),
    )(agg0, agg1, bp, Wc2p, bc2p)


def _sc_edge_layer(YP, p16, src, dst, c16, zrows):
    mesh = plsc.VectorSubcoreMesh(core_axis_name="c", subcore_axis_name="s",
                                  num_cores=NC, num_subcores=NS)
    nblk = EP // NW // BLK
    nch = nblk // CHB
    ypf = YP.reshape(NC * NPAD, YPC)

    @functools.partial(
        pl.kernel,
        out_type=jax.ShapeDtypeStruct((NC, NPAD, D2), jnp.float32),
        mesh=mesh,
        compiler_params=pltpu.CompilerParams(needs_layout_passes=False,
                                             use_tc_tiling_on_sc=False),
        scratch_types=[
            pltpu.VMEM_SHARED((NPAD, D2), jnp.float32),
            [pltpu.VMEM((CHE,), jnp.int32)] * 2,
            [pltpu.VMEM((CHE,), jnp.int32)] * 2,
            pltpu.VMEM((BLK,), jnp.int32),
            [pltpu.VMEM((CHE, 16), jnp.float32)] * 2,
            [pltpu.VMEM((BLK, YPC), jnp.float32)] * NBUF,
            pltpu.VMEM((BLK, D2), jnp.float32),
            pltpu.VMEM((16,), jnp.float32),
            pltpu.VMEM((H * 16,), jnp.float32),
            pltpu.VMEM((ZR, D2), jnp.float32),
            [pltpu.SemaphoreType.DMA] * NBUF,
            pltpu.SemaphoreType.DMA,
            pltpu.SemaphoreType.DMA,
            [pltpu.SemaphoreType.DMA] * 2,
        ],
    )
    def k(ypf_h, p_h, src_h, dst_h, c_h, z_h, out_h,
          aggs, scs, scd, didx_s, pdc, yps, mb, cv, wbuf, zb,
          semg, sems, semi, semp):
        cid = lax.axis_index("c")
        sid = lax.axis_index("s")
        wid = sid * NC + cid

        pltpu.sync_copy(z_h, zb)
        rows_per_sub = NPAD // NS
        for t in range(rows_per_sub // ZR):
            pltpu.sync_copy(zb, aggs.at[pl.ds(sid * rows_per_sub + t * ZR, ZR)])
        pltpu.sync_copy(c_h, cv)
        plsc.subcore_barrier()

        cvec = cv[...]
        iota16 = lax.iota(jnp.int32, 16)
        ones16 = jnp.where(iota16 == 0, jnp.float32(1.0), jnp.float32(0.0))
        for e in range(BLK):
            mb[e, pl.ds(DH, 16)] = ones16
        base0 = wid * (EP // NW)
        rowoff = cid * NPAD

        def issue_idx(s, ci):
            base = base0 + ci * CHE
            pltpu.async_copy(src_h.at[pl.ds(base, CHE)], scs[s], semi)
            pltpu.async_copy(dst_h.at[pl.ds(base, CHE)], scd[s], semi)

        def drain_idx(s, ci):
            base = base0 + ci * CHE
            pltpu.make_async_copy(src_h.at[pl.ds(base, CHE)], scs[s],
                                  semi).wait()
            pltpu.make_async_copy(dst_h.at[pl.ds(base, CHE)], scd[s],
                                  semi).wait()

        def issue_pd(s):
            for q in range(CHE // 128):
                pltpu.async_copy(p_h.at[scd[s].at[pl.ds(q * 128, 128)]],
                                 pdc[s].at[pl.ds(q * 128, 128)], semp[s])

        def drain_pd(s):
            for q in range(CHE // 128):
                pltpu.make_async_copy(p_h.at[scd[s].at[pl.ds(q * 128, 128)]],
                                      pdc[s].at[pl.ds(q * 128, 128)],
                                      semp[s]).wait()

        def issue(b, s, local):
            sref = scs[s].at[pl.ds(local * BLK, BLK)]
            pltpu.async_copy(ypf_h.at[sref], yps[b], semg[b])

        def drain_gathers(b, s, local):
            sref = scs[s].at[pl.ds(local * BLK, BLK)]
            pltpu.make_async_copy(ypf_h.at[sref], yps[b], semg[b]).wait()

        def drain_scatter():
            pltpu.make_async_copy(mb, aggs.at[didx_s], sems).wait()

        def consume(b, s, local, g):
            drain_gathers(b, s, local)

            qs = []
            for h in range(H):
                colp = jnp.full((16,), H * DH + h, jnp.int32)
                colq = jnp.full((16,), h, jnp.int32)
                rows = local * BLK + iota16
                qs.append(plsc.load_gather(yps[b], [iota16, colp])
                          - plsc.load_gather(pdc[s], [rows, colq])
                          + cvec[h])
            mx = qs[0]
            for h in range(1, H):
                mx = jnp.maximum(mx, qs[h])
            es = [jnp.exp(q - mx) for q in qs]
            ssum = es[0]
            for h in range(1, H):
                ssum = ssum + es[h]
            rs = 1.0 / ssum
            for h in range(H):
                wbuf[pl.ds(h * 16, 16)] = es[h] * rs

            @pl.when(g >= 1)
            def _():
                drain_scatter()
            for e in range(BLK):
                wv = [plsc.load_gather(wbuf, [jnp.full((16,), h * 16 + e,
                                                       jnp.int32)])
                      for h in range(H)]
                for j in range(DH // 16):
                    acc = wv[0] * yps[b][e, pl.ds(j * 16, 16)]
                    for h in range(1, H):
                        acc = acc + wv[h] * yps[b][e, pl.ds(h * DH + j * 16, 16)]
                    mb[e, pl.ds(j * 16, 16)] = acc
            didx_s[...] = scd[s][pl.ds(local * BLK, BLK)]
            pltpu.async_copy(mb, aggs.at[didx_s], sems, add=True)

        def chunk_body(s, ci):
            drain_idx(s, ci)
            for t in range(CHE // 16):
                scs[s][pl.ds(t * 16, 16)] = scs[s][pl.ds(t * 16, 16)] + rowoff

            @pl.when(ci + 1 < nch)
            def _():
                issue_idx((s + 1) % 2, ci + 1)
            issue_pd(s)
            for pre in range(NBUF - 1):
                issue(pre, s, pre)
            drain_pd(s)

            def quad(t, carry):
                for b in range(NBUF):
                    local = NBUF * t + b

                    @pl.when(local + NBUF - 1 < CHB)
                    def _():
                        issue((b + NBUF - 1) % NBUF, s, local + NBUF - 1)
                    consume(b, s, local, ci * CHB + local)
                return carry

            lax.fori_loop(0, CHB // NBUF, quad, 0)

        issue_idx(0, 0)

        def outer(cp, carry):
            for s in range(2):
                chunk_body(s, 2 * cp + s)
            return carry

        lax.fori_loop(0, nch // 2, outer, 0)
        drain_scatter()
        plsc.subcore_barrier()
        pltpu.sync_copy(aggs.at[pl.ds(sid * rows_per_sub, rows_per_sub)],
                        out_h.at[cid, pl.ds(sid * rows_per_sub, rows_per_sub)])

    return k(ypf, p16, src, dst, c16, zrows)


def _pad_u(u):
    return jnp.zeros((D, 16), jnp.float32).at[:, :H].set(u)


def _pad_c(c):
    return jnp.full((16,), -1e30, jnp.float32).at[:H].set(c)


def _split_w(W):
    Wr = W.reshape(D, H, D)
    Wa = Wr[:, :, :DH].reshape(D, H * DH)
    Wb = Wr[:, :, DH:].reshape(D, H * DH)
    return Wa, Wb


def kernel(pos, norm, edge_index, Wc1, bc1, W1, u1, c1, b1, W2, u2, c2, b2,
           W3, u3, c3, b3, W4, u4, c4, b4, Wc2, bc2):
    epad = jnp.full((EP - E,), NPAD - 1, jnp.int32)
    src = jnp.concatenate([edge_index[0], epad])
    dst = jnp.concatenate([edge_index[1], epad])
    zrows = jnp.zeros((ZR, D2), jnp.float32)

    xin = jnp.zeros((NPAD, D), jnp.float32)
    xin = xin.at[:N, :6].set(jnp.concatenate([pos, norm], axis=1))
    Wc1p = jnp.zeros((D, D), jnp.float32).at[:6].set(Wc1.T)
    Wa1, Wb1 = _split_w(W1)
    YP, p16 = _prologue(xin, Wc1p, bc1[None], Wa1, Wb1, _pad_u(u1))

    for (W, u, c, b) in ((W2, u2, c1, b1), (W3, u3, c2, b2), (W4, u4, c3, b3)):
        agg = _sc_edge_layer(YP, p16, src, dst, _pad_c(c), zrows)
        Wa, Wb = _split_w(W)
        YP, p16 = _combine(agg[0], agg[1], b[None], Wa, Wb, _pad_u(u))
    agg = _sc_edge_layer(YP, p16, src, dst, _pad_c(c4), zrows)

    Wc2p = jnp.zeros((D, D), jnp.float32).at[:, :3].set(Wc2.T)
    bc2p = jnp.zeros((D,), jnp.float32).at[:3].set(bc2)
    out = _epilogue(agg[0], agg[1], b4[None], Wc2p, bc2p[None])
    return out[:N, :3]
